# Initial kernel scaffold; baseline (speedup 1.0000x reference)
import jax
import jax.numpy as jnp
from jax.experimental import pallas as pl


def kernel(inputs, table, pos_table):
    raise NotImplementedError("write your pallas kernel here")


# speedup vs baseline: 3.5016x; 3.5016x over previous
import functools
import math

import jax
import jax.numpy as jnp
from jax import lax
from jax.experimental import pallas as pl
from jax.experimental.pallas import tpu as pltpu
from jax.experimental.pallas import tpu_sc as plsc

B = 4096
T = 100
D = 128
NC = 2
NS = 16
NW = NC * NS
BC = B // NW
NBUF = 2
SCALE = math.sqrt(D)
L = 16
VPR = D // L

_mesh = plsc.VectorSubcoreMesh(core_axis_name="c", subcore_axis_name="s")


@functools.partial(
    pl.kernel,
    mesh=_mesh,
    out_type=jax.ShapeDtypeStruct((B, T, D), jnp.float32),
    scratch_types=[
        pltpu.VMEM((T, BC), jnp.int32),
        pltpu.VMEM((T, D), jnp.float32),
        pltpu.VMEM((NBUF, BC, D), jnp.float32),
        pltpu.VMEM((NBUF, BC, D), jnp.float32),
        pltpu.SemaphoreType.DMA((NBUF,)),
        pltpu.SemaphoreType.DMA((NBUF,)),
    ],
)
def _emb_lookup(idx_hbm, table_hbm, pos_hbm, out_hbm,
                idx_v, pos_v, rows_v, outb_v, gsem, wsem):
    w = lax.axis_index("s") * NC + lax.axis_index("c")
    b0 = w * BC

    pltpu.sync_copy(idx_hbm.at[:, pl.ds(b0, BC)], idx_v)
    pltpu.sync_copy(pos_hbm, pos_v)

    for k in range(NBUF):
        pltpu.async_copy(table_hbm.at[idx_v.at[k]], rows_v.at[k], gsem.at[k])

    def outer(i, carry):
        for k in range(NBUF):
            t = i * NBUF + k
            pltpu.make_async_copy(
                table_hbm.at[idx_v.at[t]], rows_v.at[k], gsem.at[k]).wait()

            @pl.when(i > 0)
            def _():
                pltpu.make_async_copy(
                    outb_v.at[k], out_hbm.at[pl.ds(b0, BC), t], wsem.at[k]
                ).wait()

            pos_regs = [pos_v[t, pl.ds(j * L, L)] for j in range(VPR)]

            def row_body(r, c):
                for j in range(VPR):
                    outb_v[k, r, pl.ds(j * L, L)] = (
                        rows_v[k, r, pl.ds(j * L, L)] * SCALE + pos_regs[j]
                    )
                return c

            lax.fori_loop(0, BC, row_body, 0, unroll=2)

            pltpu.async_copy(
                outb_v.at[k], out_hbm.at[pl.ds(b0, BC), t], wsem.at[k])

            @pl.when(t + NBUF < T)
            def _():
                pltpu.async_copy(
                    table_hbm.at[idx_v.at[t + NBUF]], rows_v.at[k], gsem.at[k])
        return carry

    lax.fori_loop(0, T // NBUF, outer, 0)

    for k in range(NBUF):
        t_last = T - NBUF + k
        pltpu.make_async_copy(
            outb_v.at[k], out_hbm.at[pl.ds(b0, BC), t_last], wsem.at[k]).wait()


def kernel(inputs, table, pos_table):
    idx_t = inputs.T
    return _emb_lookup(idx_t, table, pos_table)


# speedup vs baseline: 3.6271x; 1.0358x over previous
import functools
import math

import jax
import jax.numpy as jnp
from jax import lax
from jax.experimental import pallas as pl
from jax.experimental.pallas import tpu as pltpu
from jax.experimental.pallas import tpu_sc as plsc

B = 4096
T = 100
D = 128
NC = 2
NS = 16
NW = NC * NS
BC = B // NW
TP = T
NBUF = 4
WBUF = 2
SCALE = math.sqrt(D)
L = 16
VPR = D // L

_mesh = plsc.VectorSubcoreMesh(core_axis_name="c", subcore_axis_name="s")


@functools.partial(
    pl.kernel,
    mesh=_mesh,
    out_type=jax.ShapeDtypeStruct((B, T, D), jnp.float32),
    scratch_types=[
        pltpu.VMEM((BC, TP), jnp.int32),
        pltpu.VMEM((T, D), jnp.float32),
        pltpu.VMEM((NBUF, T, D), jnp.float32),
        pltpu.VMEM((WBUF, T, D), jnp.float32),
        pltpu.SemaphoreType.DMA((NBUF,)),
        pltpu.SemaphoreType.DMA((WBUF,)),
    ],
)
def _emb_lookup(inp_hbm, table_hbm, pos_hbm, out_hbm,
                idx_v, pos_v, rows_v, outb_v, gsem, wsem):
    w = lax.axis_index("s") * NC + lax.axis_index("c")
    b0 = w * BC

    pltpu.sync_copy(inp_hbm.at[pl.ds(b0, BC)], idx_v)
    pltpu.sync_copy(pos_hbm, pos_v)

    def idx_list(q):
        return idx_v.at[q, pl.ds(0, T)]

    for k in range(NBUF):
        pltpu.async_copy(table_hbm.at[idx_list(k)], rows_v.at[k], gsem.at[k])

    def outer(i, carry):
        for k in range(NBUF):
            q = i * NBUF + k
            b = b0 + q
            kw = k % WBUF
            pltpu.make_async_copy(
                table_hbm.at[idx_list(q)], rows_v.at[k], gsem.at[k]).wait()

            def wait_write():
                pltpu.make_async_copy(
                    outb_v.at[kw], out_hbm.at[b], wsem.at[kw]).wait()

            if k < WBUF:
                pl.when(i > 0)(wait_write)
            else:
                wait_write()

            def row_body(r, c):
                for j in range(VPR):
                    sl = pl.ds(j * L, L)
                    outb_v[kw, r, sl] = rows_v[k, r, sl] * SCALE + pos_v[r, sl]
                return c

            lax.fori_loop(0, T, row_body, 0, unroll=2)

            pltpu.async_copy(outb_v.at[kw], out_hbm.at[b], wsem.at[kw])

            @pl.when(q + NBUF < BC)
            def _():
                pltpu.async_copy(
                    table_hbm.at[idx_list(q + NBUF)], rows_v.at[k], gsem.at[k])
        return carry

    lax.fori_loop(0, BC // NBUF, outer, 0)

    for k in range(WBUF):
        b_last = b0 + BC - WBUF + k
        pltpu.make_async_copy(
            outb_v.at[k], out_hbm.at[b_last], wsem.at[k]).wait()


def kernel(inputs, table, pos_table):
    return _emb_lookup(inputs, table, pos_table)


# speedup vs baseline: 3.6337x; 1.0018x over previous
import functools
import math

import jax
import jax.numpy as jnp
from jax import lax
from jax.experimental import pallas as pl
from jax.experimental.pallas import tpu as pltpu
from jax.experimental.pallas import tpu_sc as plsc

B = 4096
T = 100
D = 128
NC = 2
NS = 16
NW = NC * NS
BC = B // NW
TP = T
NBUF = 4
WBUF = 2
SCALE = math.sqrt(D)
L = 16
VPR = D // L

_mesh = plsc.VectorSubcoreMesh(core_axis_name="c", subcore_axis_name="s")


@functools.partial(
    pl.kernel,
    mesh=_mesh,
    compiler_params=pltpu.CompilerParams(use_tc_tiling_on_sc=True),
    out_type=jax.ShapeDtypeStruct((B, T, D), jnp.float32),
    scratch_types=[
        pltpu.VMEM((BC, TP), jnp.int32),
        pltpu.VMEM((T, D), jnp.float32),
        pltpu.VMEM((NBUF, T, D), jnp.float32),
        pltpu.VMEM((WBUF, T, D), jnp.float32),
        pltpu.SemaphoreType.DMA((NBUF,)),
        pltpu.SemaphoreType.DMA((WBUF,)),
    ],
)
def _emb_lookup(inp_hbm, table_hbm, pos_hbm, out_hbm,
                idx_v, pos_v, rows_v, outb_v, gsem, wsem):
    w = lax.axis_index("s") * NC + lax.axis_index("c")
    b0 = w * BC

    pltpu.sync_copy(inp_hbm.at[pl.ds(b0, BC)], idx_v)
    pltpu.sync_copy(pos_hbm, pos_v)

    def idx_list(q):
        return idx_v.at[q, pl.ds(0, T)]

    for k in range(NBUF):
        pltpu.async_copy(table_hbm.at[idx_list(k)], rows_v.at[k], gsem.at[k])

    def outer(i, carry):
        for k in range(NBUF):
            q = i * NBUF + k
            b = b0 + q
            kw = k % WBUF
            pltpu.make_async_copy(
                table_hbm.at[idx_list(q)], rows_v.at[k], gsem.at[k]).wait()

            def wait_write():
                pltpu.make_async_copy(
                    outb_v.at[kw], out_hbm.at[b], wsem.at[kw]).wait()

            if k < WBUF:
                pl.when(i > 0)(wait_write)
            else:
                wait_write()

            def row_body(r, c):
                for j in range(VPR):
                    sl = pl.ds(j * L, L)
                    outb_v[kw, r, sl] = rows_v[k, r, sl] * SCALE + pos_v[r, sl]
                return c

            lax.fori_loop(0, T, row_body, 0, unroll=2)

            pltpu.async_copy(outb_v.at[kw], out_hbm.at[b], wsem.at[kw])

            @pl.when(q + NBUF < BC)
            def _():
                pltpu.async_copy(
                    table_hbm.at[idx_list(q + NBUF)], rows_v.at[k], gsem.at[k])
        return carry

    lax.fori_loop(0, BC // NBUF, outer, 0)

    for k in range(WBUF):
        b_last = b0 + BC - WBUF + k
        pltpu.make_async_copy(
            outb_v.at[k], out_hbm.at[b_last], wsem.at[k]).wait()


def kernel(inputs, table, pos_table):
    return _emb_lookup(inputs, table, pos_table)
